# baseline (speedup 1.0000x reference)
import functools

import numpy as np
import jax
import jax.numpy as jnp
from jax import lax
from jax.experimental import pallas as pl
from jax.experimental.pallas import tpu as pltpu
from jax.experimental.pallas import tpu_sc as plsc

ROWS = 64
COLS = 32768
LANES = 16
NUM_CORES = 2
NUM_SUBCORES = 16
NUM_WORKERS = NUM_CORES * NUM_SUBCORES
ROWS_PER_WORKER = ROWS // NUM_WORKERS
UNROLL_A = 8
UNROLL_B = 4
NBLK = 16
BLK = COLS // NBLK
BLK_CHUNKS = BLK // LANES
BIG = 2**30


def _row_argmax(row_ref, bmax_ref):
  lane_iota = lax.iota(jnp.int32, LANES)
  neg_inf_v = jnp.full((LANES,), -np.inf, jnp.float32)

  def outer(b, carry):
    def inner(g, maxs):
      maxs = list(maxs)
      base = b * BLK + g * (UNROLL_A * LANES)
      for u in range(UNROLL_A):
        v = row_ref[pl.ds(base + u * LANES, LANES)]
        maxs[u] = jnp.maximum(maxs[u], v)
      return tuple(maxs)

    maxs = lax.fori_loop(0, BLK_CHUNKS // UNROLL_A, inner,
                         tuple(neg_inf_v for _ in range(UNROLL_A)))
    m = maxs[0]
    for u in range(1, UNROLL_A):
      m = jnp.maximum(m, maxs[u])
    bmax_ref[pl.ds(b * LANES, LANES)] = m
    return carry

  lax.fori_loop(0, NBLK, outer, jnp.int32(0))

  bmaxes = [bmax_ref[pl.ds(b * LANES, LANES)] for b in range(NBLK)]
  gv = bmaxes[0]
  for b in range(1, NBLK):
    gv = jnp.maximum(gv, bmaxes[b])
  gmax = jnp.max(gv, axis=0)
  bv = jnp.full((LANES,), BIG, jnp.int32)
  for b in range(NBLK - 1, -1, -1):
    bv = jnp.where(bmaxes[b] == gmax, b, bv)
  bmin = jnp.min(bv, axis=0)

  base0 = bmin * BLK

  def bodyb(g, bests):
    bests = list(bests)
    base = base0 + g * (UNROLL_B * LANES)
    for u in range(UNROLL_B):
      v = row_ref[pl.ds(base + u * LANES, LANES)]
      cand = lane_iota + (base + u * LANES)
      bests[u] = jnp.minimum(bests[u], jnp.where(v == gmax, cand, BIG))
    return tuple(bests)

  bests = lax.fori_loop(0, BLK_CHUNKS // UNROLL_B, bodyb,
                        tuple(jnp.full((LANES,), BIG, jnp.int32)
                              for _ in range(UNROLL_B)))
  best = bests[0]
  for u in range(1, UNROLL_B):
    best = jnp.minimum(best, bests[u])
  return jnp.min(best, axis=0)


def _body(x_hbm, out_hbm, row0_v, row1_v, bmax_v, res_v, sem0, sem1):
  cid = lax.axis_index("c")
  sid = lax.axis_index("s")
  wid = cid * NUM_SUBCORES + sid
  r0 = wid * ROWS_PER_WORKER

  cp0 = pltpu.make_async_copy(x_hbm.at[r0], row0_v, sem0)
  cp0.start()
  cp1 = pltpu.make_async_copy(x_hbm.at[r0 + 1], row1_v, sem1)
  cp1.start()

  cp0.wait()
  a0 = _row_argmax(row0_v, bmax_v)
  cp1.wait()
  a1 = _row_argmax(row1_v, bmax_v)

  lane_iota = lax.iota(jnp.int32, LANES)
  res_v[...] = jnp.where(lane_iota == 0, a0, a1)
  pltpu.sync_copy(res_v, out_hbm.at[wid])


@jax.jit
def kernel(x):
  mesh = plsc.VectorSubcoreMesh(
      core_axis_name="c", subcore_axis_name="s",
      num_cores=NUM_CORES, num_subcores=NUM_SUBCORES)
  padded = pl.kernel(
      _body,
      out_type=jax.ShapeDtypeStruct((NUM_WORKERS, LANES), jnp.int32),
      mesh=mesh,
      scratch_types=[
          pltpu.VMEM((COLS,), jnp.float32),
          pltpu.VMEM((COLS,), jnp.float32),
          pltpu.VMEM((NBLK * LANES,), jnp.float32),
          pltpu.VMEM((LANES,), jnp.int32),
          pltpu.SemaphoreType.DMA,
          pltpu.SemaphoreType.DMA,
      ],
      compiler_params=pltpu.CompilerParams(
          needs_layout_passes=False,
          disable_bounds_checks=True,
          disable_semaphore_checks=True,
      ),
  )(x)
  return padded[:, :ROWS_PER_WORKER].reshape(ROWS)
